# baseline (speedup 1.0000x reference)
import math

import jax
import jax.numpy as jnp
from jax.experimental import pallas as pl
from jax.experimental.pallas import tpu as pltpu

B = 16
H = 8
D = 128
BS = 16
CTX = 1024
NUM_BLOCKS = 1024
MAXB = 64
KV_SCALE = 1.0
ROPE_BASE = 10000.0
HALF = D // 2
SCALE = 1.0 / math.sqrt(D)

CH = 8
T = CH * BS

_CONTRACT_MINOR = (((1,), (1,)), ((), ()))
_CONTRACT_PV = (((1,), (0,)), ((), ()))


def _inv_freq_row():
  fidx = jax.lax.broadcasted_iota(jnp.int32, (1, HALF), 1).astype(jnp.float32)
  return 1.0 / (ROPE_BASE ** (fidx / HALF))


def _rot_coeffs(pos_f32_col):
  ang = pos_f32_col * _inv_freq_row()
  c = jnp.cos(ang)
  s = jnp.sin(ang)
  return jnp.concatenate([c, c], axis=-1), jnp.concatenate([-s, s], axis=-1)


def _swap_halves(x):
  return jnp.concatenate([x[..., HALF:], x[..., :HALF]], axis=-1)


def _attn_body(bt_ref, sl_ref, q_ref, k_ref, v_ref, kc_ref, vc_ref, o_ref,
               kbuf, vbuf, ksem, vsem):
  i = pl.program_id(0)

  s = 257 + sl_ref[i] % (2048 - 257)
  num_past = s - 1
  rem = num_past % BS
  within = num_past < CTX
  full = jnp.where(within, num_past // BS, (CTX // BS) - 1)
  n_valid = full * BS + rem
  nblocks = (n_valid + BS - 1) // BS
  nchunks = (nblocks + CH - 1) // CH

  def copies(c, slot):
    out = []
    for b in range(CH):
      safe = jnp.minimum(c * CH + b, nblocks - 1)
      bt = bt_ref[i, safe]
      out.append(pltpu.make_async_copy(
          kc_ref.at[bt], kbuf.at[slot, pl.ds(b * BS, BS)], ksem.at[slot]))
      out.append(pltpu.make_async_copy(
          vc_ref.at[bt], vbuf.at[slot, pl.ds(b * BS, BS)], vsem.at[slot]))
    return out

  def issue(c, slot):
    for cp in copies(c, slot):
      cp.start()

  def wait(c, slot):
    for cp in copies(c, slot):
      cp.wait()

  issue(0, 0)

  cur_pos = jnp.minimum(num_past, CTX - 1)
  qC, qS = _rot_coeffs(jnp.full((1, 1), cur_pos, jnp.float32))
  qh = q_ref[0]
  kh = k_ref[0]
  q_rot = qh * qC + _swap_halves(qh) * qS
  k_rot = kh * qC + _swap_halves(kh) * qS

  jt_col = jax.lax.broadcasted_iota(jnp.int32, (T, 1), 0)
  jt_lane = jax.lax.broadcasted_iota(jnp.int32, (1, T), 1)

  def chunk_body(c, carry):
    ms, ls, accs = carry
    slot = jax.lax.rem(c, 2)

    @pl.when(c + 1 < nchunks)
    def _():
      issue(c + 1, 1 - slot)

    wait(c, slot)

    jc = c * T + jt_col
    pos = jnp.where(within, jc,
                    jnp.where(jc < BS, jc, jc + BS - 1 - rem)).astype(jnp.float32)
    C, S = _rot_coeffs(pos)
    mask = (c * T + jt_lane) < n_valid

    ms_n, ls_n, accs_n = [], [], []
    for h in range(H):
      Xh = kbuf[slot, :, h, :]
      Xr = Xh * C + _swap_halves(Xh) * S
      qr = q_rot[h:h + 1, :]
      sc = jax.lax.dot_general(qr, Xr, _CONTRACT_MINOR,
                               preferred_element_type=jnp.float32) * SCALE
      sc = jnp.where(mask, sc, -1e30)
      m_c = jnp.max(sc)
      m_new = jnp.maximum(ms[h], m_c)
      alpha = jnp.exp(ms[h] - m_new)
      p = jnp.where(mask, jnp.exp(sc - m_new), 0.0)
      l_new = alpha * ls[h] + jnp.sum(p)
      Vh = vbuf[slot, :, h, :]
      pv = jax.lax.dot_general(p, Vh, _CONTRACT_PV,
                               preferred_element_type=jnp.float32)
      acc_new = alpha * accs[h] + pv
      ms_n.append(m_new)
      ls_n.append(l_new)
      accs_n.append(acc_new)
    return tuple(ms_n), tuple(ls_n), tuple(accs_n)

  neg = jnp.float32(-1e30)
  m0 = tuple(neg for _ in range(H))
  l0 = tuple(jnp.float32(0.0) for _ in range(H))
  a0 = tuple(jnp.zeros((1, D), jnp.float32) for _ in range(H))
  ms, ls, accs = jax.lax.fori_loop(0, nchunks, chunk_body, (m0, l0, a0))

  for h in range(H):
    s_cur = jnp.sum(q_rot[h:h + 1, :] * k_rot[h:h + 1, :]) * SCALE
    m_f = jnp.maximum(ms[h], s_cur)
    alpha = jnp.exp(ms[h] - m_f)
    p_cur = jnp.exp(s_cur - m_f)
    l_f = alpha * ls[h] + p_cur
    out_h = (alpha * accs[h] + p_cur * v_ref[0, h:h + 1, :]) / l_f
    o_ref[0, h:h + 1, :] = out_h


@jax.jit
def kernel(q, k, v, key_cache, value_cache, block_tables, seq_lens, positions):
  del positions
  grid_spec = pltpu.PrefetchScalarGridSpec(
      num_scalar_prefetch=2,
      grid=(B,),
      in_specs=[
          pl.BlockSpec((1, H, D), lambda i, bt, sl: (i, 0, 0)),
          pl.BlockSpec((1, H, D), lambda i, bt, sl: (i, 0, 0)),
          pl.BlockSpec((1, H, D), lambda i, bt, sl: (i, 0, 0)),
          pl.BlockSpec(memory_space=pl.MemorySpace.ANY),
          pl.BlockSpec(memory_space=pl.MemorySpace.ANY),
      ],
      out_specs=pl.BlockSpec((1, H, D), lambda i, bt, sl: (i, 0, 0)),
      scratch_shapes=[
          pltpu.VMEM((2, T, H, D), jnp.float32),
          pltpu.VMEM((2, T, H, D), jnp.float32),
          pltpu.SemaphoreType.DMA((2,)),
          pltpu.SemaphoreType.DMA((2,)),
      ],
  )
  out = pl.pallas_call(
      _attn_body,
      grid_spec=grid_spec,
      out_shape=jax.ShapeDtypeStruct((B, H, D), jnp.float32),
  )(block_tables, seq_lens, q.reshape(B, H, D), k.reshape(B, H, D),
    v.reshape(B, H, D), key_cache, value_cache)
  return out.reshape(B, H * D)


# speedup vs baseline: 1.1020x; 1.1020x over previous
import math

import jax
import jax.numpy as jnp
from jax.experimental import pallas as pl
from jax.experimental.pallas import tpu as pltpu

B = 16
H = 8
D = 128
BS = 16
CTX = 1024
NUM_BLOCKS = 1024
MAXB = 64
KV_SCALE = 1.0
ROPE_BASE = 10000.0
HALF = D // 2
SCALE = 1.0 / math.sqrt(D)

CH = 8
T = CH * BS

_DOT_MINOR = (((1,), (0,)), ((), ()))


def _swap_halves(x):
  return jnp.concatenate([x[..., HALF:], x[..., :HALF]], axis=-1)


def _attn_body(bt_ref, sl_ref, q_ref, k_ref, v_ref, kc_ref, vc_ref, o_ref,
               kbuf, vbuf, ksem, vsem):
  i = pl.program_id(0)

  s = 257 + sl_ref[i] % (2048 - 257)
  num_past = s - 1
  rem = num_past % BS
  within = num_past < CTX
  full = jnp.where(within, num_past // BS, (CTX // BS) - 1)
  n_valid = full * BS + rem
  nblocks = (n_valid + BS - 1) // BS
  nchunks = (nblocks + CH - 1) // CH

  def copies(c, slot):
    out = []
    for b in range(CH):
      safe = jnp.minimum(c * CH + b, nblocks - 1)
      bt = bt_ref[i, safe]
      out.append(pltpu.make_async_copy(
          kc_ref.at[bt], kbuf.at[slot, pl.ds(b * BS, BS)], ksem.at[slot]))
      out.append(pltpu.make_async_copy(
          vc_ref.at[bt], vbuf.at[slot, pl.ds(b * BS, BS)], vsem.at[slot]))
    return out

  def issue(c, slot):
    for cp in copies(c, slot):
      cp.start()

  def wait(c, slot):
    for cp in copies(c, slot):
      cp.wait()

  issue(0, 0)

  inv3 = ROPE_BASE ** (
      -jax.lax.broadcasted_iota(jnp.int32, (1, H, HALF), 2).astype(jnp.float32)
      / HALF)
  jt3 = jax.lax.broadcasted_iota(jnp.int32, (T, H, HALF), 0)
  ang_base3 = jt3.astype(jnp.float32) * inv3
  shift_inv3 = ((BS - 1 - rem).astype(jnp.float32) * inv3)
  jt1 = jax.lax.broadcasted_iota(jnp.int32, (T, 1, 1), 0)
  ones_col = jnp.ones((D, 1), jnp.float32)
  beyond = jnp.logical_not(within)

  cur_pos = jnp.minimum(num_past, CTX - 1).astype(jnp.float32)
  qc = jnp.cos(cur_pos * inv3)
  qs = jnp.sin(cur_pos * inv3)
  qC = jnp.concatenate([qc, qc], axis=2)
  qS = jnp.concatenate([-qs, qs], axis=2)
  qh = q_ref[...]
  kh = k_ref[...]
  q_rot = qh * qC + _swap_halves(qh) * qS
  k_rot = kh * qC + _swap_halves(kh) * qS

  def chunk_body(c, carry):
    m, l, acc = carry
    slot = jax.lax.rem(c, 2)

    @pl.when(c + 1 < nchunks)
    def _():
      issue(c + 1, 1 - slot)

    wait(c, slot)

    sel3 = jnp.logical_and(beyond, (c * T + jt3) >= BS)
    ang3 = (ang_base3 + (c * T).astype(jnp.float32) * inv3
            + jnp.where(sel3, shift_inv3, 0.0))
    c3 = jnp.cos(ang3)
    s3 = jnp.sin(ang3)
    C3 = jnp.concatenate([c3, c3], axis=2)
    S3 = jnp.concatenate([-s3, s3], axis=2)
    mask3 = (c * T + jt1) < n_valid

    X3 = kbuf[slot]
    Xr3 = X3 * C3 + _swap_halves(X3) * S3
    t3 = Xr3 * q_rot
    sc = jax.lax.dot_general(t3.reshape(T * H, D), ones_col, _DOT_MINOR,
                             preferred_element_type=jnp.float32)
    sc3 = sc.reshape(T, H, 1) * SCALE
    sc3 = jnp.where(mask3, sc3, -1e30)
    m_c = jnp.max(sc3, axis=0, keepdims=True)
    m_new = jnp.maximum(m, m_c)
    alpha = jnp.exp(m - m_new)
    p3 = jnp.exp(sc3 - m_new)
    l_new = alpha * l + jnp.sum(p3, axis=0, keepdims=True)
    W3 = vbuf[slot] * p3
    pv = jnp.sum(W3, axis=0, keepdims=True)
    acc_new = alpha * acc + pv
    return m_new, l_new, acc_new

  m0 = jnp.full((1, H, 1), -1e30, jnp.float32)
  l0 = jnp.zeros((1, H, 1), jnp.float32)
  a0 = jnp.zeros((1, H, D), jnp.float32)
  m, l, acc = jax.lax.fori_loop(0, nchunks, chunk_body, (m0, l0, a0))

  t_cur = q_rot * k_rot
  s_cur = jax.lax.dot_general(t_cur.reshape(H, D), ones_col, _DOT_MINOR,
                              preferred_element_type=jnp.float32)
  s_cur3 = s_cur.reshape(1, H, 1) * SCALE
  m_f = jnp.maximum(m, s_cur3)
  alpha = jnp.exp(m - m_f)
  p_cur = jnp.exp(s_cur3 - m_f)
  l_f = alpha * l + p_cur
  o_ref[...] = (alpha * acc + p_cur * v_ref[...]) / l_f


@jax.jit
def kernel(q, k, v, key_cache, value_cache, block_tables, seq_lens, positions):
  del positions
  grid_spec = pltpu.PrefetchScalarGridSpec(
      num_scalar_prefetch=2,
      grid=(B,),
      in_specs=[
          pl.BlockSpec((1, H, D), lambda i, bt, sl: (i, 0, 0)),
          pl.BlockSpec((1, H, D), lambda i, bt, sl: (i, 0, 0)),
          pl.BlockSpec((1, H, D), lambda i, bt, sl: (i, 0, 0)),
          pl.BlockSpec(memory_space=pl.MemorySpace.ANY),
          pl.BlockSpec(memory_space=pl.MemorySpace.ANY),
      ],
      out_specs=pl.BlockSpec((1, H, D), lambda i, bt, sl: (i, 0, 0)),
      scratch_shapes=[
          pltpu.VMEM((2, T, H, D), jnp.float32),
          pltpu.VMEM((2, T, H, D), jnp.float32),
          pltpu.SemaphoreType.DMA((2,)),
          pltpu.SemaphoreType.DMA((2,)),
      ],
  )
  out = pl.pallas_call(
      _attn_body,
      grid_spec=grid_spec,
      out_shape=jax.ShapeDtypeStruct((B, H, D), jnp.float32),
  )(block_tables, seq_lens, q.reshape(B, H, D), k.reshape(B, H, D),
    v.reshape(B, H, D), key_cache, value_cache)
  return out.reshape(B, H * D)


# speedup vs baseline: 2.0681x; 1.8766x over previous
import math

import jax
import jax.numpy as jnp
import numpy as np
from jax.experimental import pallas as pl
from jax.experimental.pallas import tpu as pltpu

B = 16
H = 8
D = 128
BS = 16
CTX = 1024
NUM_BLOCKS = 1024
MAXB = 64
KV_SCALE = 1.0
ROPE_BASE = 10000.0
HALF = D // 2
SCALE = 1.0 / math.sqrt(D)

CH = 8
T = CH * BS
NCH = (MAXB * BS) // T

_DOT_MINOR = (((1,), (0,)), ((), ()))

_inv = (ROPE_BASE ** (-np.arange(HALF) / HALF))[None, :]


def _ctab(pos_col):
  ang = pos_col[:, None, :] * _inv[None]
  n = ang.shape[0]
  c = np.broadcast_to(np.cos(ang), (n, H, HALF)).astype(np.float32).copy()
  s = np.broadcast_to(np.sin(ang), (n, H, HALF)).astype(np.float32).copy()
  return c, s

_BASE_C, _BASE_S = _ctab(np.arange(T)[:, None].astype(np.float64))
_DELTA_C, _DELTA_S = _ctab((np.arange(NCH) * T)[:, None].astype(np.float64))
_SHIFT_C, _SHIFT_S = _ctab((BS - 1 - np.arange(BS))[:, None].astype(np.float64))


def _attn_body(bt_ref, sl_ref, q_ref, k_ref, v_ref,
               bC_ref, bS_ref, dC_ref, dS_ref, hC_ref, hS_ref,
               kc_ref, vc_ref, o_ref, kbuf, vbuf, ksem, vsem):
  i = pl.program_id(0)

  s = 257 + sl_ref[i] % (2048 - 257)
  num_past = s - 1
  rem = num_past % BS
  within = num_past < CTX
  full = jnp.where(within, num_past // BS, (CTX // BS) - 1)
  n_valid = full * BS + rem
  nblocks = (n_valid + BS - 1) // BS
  nchunks = (nblocks + CH - 1) // CH

  def copies(c, slot):
    out = []
    for b in range(CH):
      safe = jnp.minimum(c * CH + b, nblocks - 1)
      bt = bt_ref[i, safe]
      out.append(pltpu.make_async_copy(
          kc_ref.at[bt], kbuf.at[slot, pl.ds(b * BS, BS)], ksem.at[slot]))
      out.append(pltpu.make_async_copy(
          vc_ref.at[bt], vbuf.at[slot, pl.ds(b * BS, BS)], vsem.at[slot]))
    return out

  def issue(c, slot):
    for cp in copies(c, slot):
      cp.start()

  def wait(c, slot):
    for cp in copies(c, slot):
      cp.wait()

  issue(0, 0)

  baseC = bC_ref[...]
  baseS = bS_ref[...]
  shc = hC_ref[rem]
  shs = hS_ref[rem]
  sbC = baseC * shc - baseS * shs
  sbS = baseS * shc + baseC * shs
  beyond = jnp.logical_not(within)
  jt3 = jax.lax.broadcasted_iota(jnp.int32, (T, H, HALF), 0)
  sink0 = jnp.logical_and(beyond, jt3 >= BS)
  b0C = jnp.where(sink0, sbC, baseC)
  b0S = jnp.where(sink0, sbS, baseS)
  bNC = jnp.where(beyond, sbC, baseC)
  bNS = jnp.where(beyond, sbS, baseS)
  jt1 = jax.lax.broadcasted_iota(jnp.int32, (T, 1, 1), 0)
  ones_half = jnp.ones((HALF, 1), jnp.float32)

  cur_pos = jnp.minimum(num_past, CTX - 1)
  cq = cur_pos // T
  rq = cur_pos % T
  qdc = dC_ref[cq]
  qds = dS_ref[cq]
  qbc = bC_ref[rq]
  qbs = bS_ref[rq]
  qcc = qbc * qdc - qbs * qds
  qss = qbs * qdc + qbc * qds
  def _rot2(x_ref):
    x1 = x_ref[0, :, :HALF]
    x2 = x_ref[0, :, HALF:]
    return x1 * qcc - x2 * qss, x2 * qcc + x1 * qss
  q1, q2 = _rot2(q_ref)
  k1, k2 = _rot2(k_ref)

  def chunk_body(c, carry):
    m, l, acc = carry
    slot = jax.lax.rem(c, 2)

    @pl.when(c + 1 < nchunks)
    def _():
      issue(c + 1, 1 - slot)

    wait(c, slot)

    is0 = c == 0
    bpC = jnp.where(is0, b0C, bNC)
    bpS = jnp.where(is0, b0S, bNS)
    dc = dC_ref[c]
    ds = dS_ref[c]
    PC = bpC * dc - bpS * ds
    PS = bpS * dc + bpC * ds
    mask3 = (c * T + jt1) < n_valid

    X = kbuf[slot]
    x1 = X[..., :HALF]
    x2 = X[..., HALF:]
    r1 = x1 * PC - x2 * PS
    r2 = x2 * PC + x1 * PS
    t3 = r1 * q1 + r2 * q2
    sc = jax.lax.dot_general(t3.reshape(T * H, HALF), ones_half, _DOT_MINOR,
                             preferred_element_type=jnp.float32)
    sc3 = sc.reshape(T, H, 1) * SCALE
    sc3 = jnp.where(mask3, sc3, -1e30)
    m_c = jnp.max(sc3, axis=0, keepdims=True)
    m_new = jnp.maximum(m, m_c)
    alpha = jnp.exp(m - m_new)
    p3 = jnp.exp(sc3 - m_new)
    l_new = alpha * l + jnp.sum(p3, axis=0, keepdims=True)
    W3 = vbuf[slot] * p3
    pv = jnp.sum(W3, axis=0, keepdims=True)
    acc_new = alpha * acc + pv
    return m_new, l_new, acc_new

  m0 = jnp.full((1, H, 1), -1e30, jnp.float32)
  l0 = jnp.zeros((1, H, 1), jnp.float32)
  a0 = jnp.zeros((1, H, D), jnp.float32)
  m, l, acc = jax.lax.fori_loop(0, nchunks, chunk_body, (m0, l0, a0))

  t_cur = q1 * k1 + q2 * k2
  s_cur = jax.lax.dot_general(t_cur, ones_half, _DOT_MINOR,
                              preferred_element_type=jnp.float32)
  s_cur3 = s_cur.reshape(1, H, 1) * SCALE
  m_f = jnp.maximum(m, s_cur3)
  alpha = jnp.exp(m - m_f)
  p_cur = jnp.exp(s_cur3 - m_f)
  l_f = alpha * l + p_cur
  o_ref[...] = (alpha * acc + p_cur * v_ref[...]) / l_f


@jax.jit
def kernel(q, k, v, key_cache, value_cache, block_tables, seq_lens, positions):
  del positions
  whole = lambda shape: pl.BlockSpec(shape, lambda i, bt, sl: (0,) * len(shape))
  row = pl.BlockSpec((1, H, D), lambda i, bt, sl: (i, 0, 0))
  grid_spec = pltpu.PrefetchScalarGridSpec(
      num_scalar_prefetch=2,
      grid=(B,),
      in_specs=[
          row, row, row,
          whole((T, H, HALF)), whole((T, H, HALF)),
          whole((NCH, H, HALF)), whole((NCH, H, HALF)),
          whole((BS, H, HALF)), whole((BS, H, HALF)),
          pl.BlockSpec(memory_space=pl.MemorySpace.ANY),
          pl.BlockSpec(memory_space=pl.MemorySpace.ANY),
      ],
      out_specs=row,
      scratch_shapes=[
          pltpu.VMEM((2, T, H, D), jnp.float32),
          pltpu.VMEM((2, T, H, D), jnp.float32),
          pltpu.SemaphoreType.DMA((2,)),
          pltpu.SemaphoreType.DMA((2,)),
      ],
  )
  out = pl.pallas_call(
      _attn_body,
      grid_spec=grid_spec,
      out_shape=jax.ShapeDtypeStruct((B, H, D), jnp.float32),
  )(block_tables, seq_lens, q.reshape(B, H, D), k.reshape(B, H, D),
    v.reshape(B, H, D),
    jnp.asarray(_BASE_C), jnp.asarray(_BASE_S),
    jnp.asarray(_DELTA_C), jnp.asarray(_DELTA_S),
    jnp.asarray(_SHIFT_C), jnp.asarray(_SHIFT_S),
    key_cache, value_cache)
  return out.reshape(B, H * D)
